# Initial kernel scaffold; baseline (speedup 1.0000x reference)
import jax
import jax.numpy as jnp
from jax.experimental import pallas as pl


def kernel(x, edge_index, batch, W0, b0, g0, bt0, W1, b1, g1, bt1, W2, b2, g2, bt2, W3, b3, g3, bt3, Wout, bout):
    raise NotImplementedError("write your pallas kernel here")


# speedup vs baseline: 2.3759x; 2.3759x over previous
import functools

import jax
import jax.numpy as jnp
from jax import lax
from jax.experimental import pallas as pl
from jax.experimental.pallas import tpu as pltpu
from jax.experimental.pallas import tpu_sc as plsc

F32 = jnp.float32
I32 = jnp.int32

N_CORES = 2
N_TILES = 16
EB = 128
SL = 128
R = 256
EPS = 1e-5
G = 64


def _split_store(o_ref, res, n_slices):
    for k in range(n_slices):
        o_ref[k] = res[:, k * SL:(k + 1) * SL]


def _cat_slices(q_ref, n_slices):
    return jnp.concatenate([q_ref[k] for k in range(n_slices)], axis=-1)


def _scale_body(n_slices, x_ref, dinv_ref, o_ref):
    _split_store(o_ref, x_ref[...] * dinv_ref[:, :1], n_slices)


def _mm_body(n_in, n_out, p_ref, w_ref, o_ref):
    h = _cat_slices(p_ref, n_in)
    res = jnp.dot(h, w_ref[...], preferred_element_type=F32)
    _split_store(o_ref, res, n_out)


def _stats_body(n_slices, q_ref, dinv_ref, o_ref):
    @pl.when(pl.program_id(0) == 0)
    def _():
        o_ref[...] = jnp.zeros_like(o_ref)

    c = _cat_slices(q_ref, n_slices) * dinv_ref[:, :1]
    s1 = jnp.sum(c, axis=0)
    s2 = jnp.sum(c * c, axis=0)
    o_ref[...] += jnp.concatenate([s1[None], s2[None]], axis=0)


def _gelu(h):
    return h * 0.5 * (1.0 + lax.erf(h * 0.7071067811865476))


def _bn_from_stats(c, stats_ref, g_ref, bt_ref, n_real):
    mean = stats_ref[0:1, :] / n_real
    var = stats_ref[1:2, :] / n_real - mean * mean
    xh = (c - mean) * lax.rsqrt(var + EPS)
    return xh * g_ref[...] + bt_ref[...]


def _bn_mm_body(n_real, n_in, n_out, q_ref, dinv_ref, stats_ref, g_ref,
                bt_ref, w_ref, o_ref):
    c = _cat_slices(q_ref, n_in) * dinv_ref[:, :1]
    h = _gelu(_bn_from_stats(c, stats_ref, g_ref, bt_ref, n_real))
    hp = h * dinv_ref[:, :1]
    res = jnp.dot(hp, w_ref[...], preferred_element_type=F32)
    _split_store(o_ref, res, n_out)


def _final_body(n_real, n_blocks, n_in, q_ref, dinv_ref, stats_ref, g_ref,
                bt_ref, batch_ref, wv_ref, bout_ref, o_ref, pool_acc):
    i = pl.program_id(0)

    @pl.when(i == 0)
    def _():
        pool_acc[...] = jnp.zeros_like(pool_acc)

    c = _cat_slices(q_ref, n_in) * dinv_ref[:, :1]
    h = _gelu(_bn_from_stats(c, stats_ref, g_ref, bt_ref, n_real))
    b = batch_ref[0]
    gids = lax.broadcasted_iota(I32, (G, R), 0)
    oh = (gids == b).astype(F32)
    pool_acc[...] += jnp.dot(oh, h, preferred_element_type=F32)

    @pl.when(i == n_blocks - 1)
    def _():
        fin = jnp.sum(pool_acc[...] * wv_ref[...], axis=1, keepdims=True)
        fin = fin + bout_ref[0:1, 0:1]
        o_ref[...] = jnp.broadcast_to(fin, (G, 128))


def _make_prop(n_pad, n_slices, f_pad):
    rpt = n_pad // N_TILES
    spc = n_slices // N_CORES
    ept = f_pad // N_TILES
    nb = ept // EB
    mesh = plsc.VectorSubcoreMesh(core_axis_name="c", subcore_axis_name="s",
                                  num_cores=N_CORES, num_subcores=N_TILES)

    @functools.partial(
        pl.kernel,
        out_type=jax.ShapeDtypeStruct((n_slices, n_pad, SL), F32),
        mesh=mesh,
        scratch_types=[
            pltpu.VMEM((EB,), I32),
            pltpu.VMEM((EB,), I32),
            pltpu.VMEM((EB, SL), F32),
            pltpu.VMEM_SHARED((n_pad, SL), F32),
            pltpu.SemaphoreType.DMA,
        ],
    )
    def prop(xw, srcg, dstg, zrows, out, sidx_v, didx_v, rows_v, acc, sem):
        core = lax.axis_index("c")
        tid = lax.axis_index("s")
        for j in range(spc):
            s = core * spc + j
            pltpu.sync_copy(zrows, acc.at[pl.ds(tid * rpt, rpt)])
            plsc.subcore_barrier()

            def body(b, _):
                base = pl.multiple_of(tid * ept + b * EB, 8)
                pltpu.sync_copy(srcg.at[pl.ds(base, EB)], sidx_v)
                pltpu.sync_copy(dstg.at[pl.ds(base, EB)], didx_v)
                pltpu.async_copy(xw.at[s].at[sidx_v], rows_v, sem).wait()
                pltpu.sync_copy(rows_v, acc.at[didx_v], add=True)
                return 0

            lax.fori_loop(0, nb, body, 0)
            plsc.subcore_barrier()
            pltpu.sync_copy(acc.at[pl.ds(tid * rpt, rpt)],
                            out.at[s].at[pl.ds(tid * rpt, rpt)])
            plsc.subcore_barrier()

    return prop


def _row_call(body, out_sds, out_spec, args_specs, grid, scratch=None):
    in_specs = [s for _, s in args_specs]
    args = [a for a, _ in args_specs]
    return pl.pallas_call(
        body, grid=grid, in_specs=in_specs, out_specs=out_spec,
        out_shape=out_sds, scratch_shapes=scratch or [],
    )(*args)


def kernel(x, edge_index, batch, W0, b0, g0, bt0, W1, b1, g1, bt1,
           W2, b2, g2, bt2, W3, b3, g3, bt3, Wout, bout):
    N, IN_DIM = x.shape
    H = W0.shape[1]
    E = edge_index.shape[1]
    F = E + N
    N_PAD = ((N + 1023) // 1024) * 1024
    F_PAD = ((F + N_TILES * EB - 1) // (N_TILES * EB)) * (N_TILES * EB)
    ZROW = N
    NS_IN = IN_DIM // SL
    NS_H = H // SL

    src = edge_index[0].astype(I32)
    dst = edge_index[1].astype(I32)
    loop = jnp.arange(N, dtype=I32)
    srcg = jnp.concatenate(
        [src, loop, jnp.full((F_PAD - F,), ZROW, I32)])
    dstg = jnp.concatenate(
        [dst, loop, jnp.full((F_PAD - F,), ZROW, I32)])
    deg = jnp.zeros((N_PAD,), F32).at[dstg].add(1.0)
    deg = deg.at[ZROW].set(0.0)
    dinv = jnp.where(deg > 0, lax.rsqrt(jnp.maximum(deg, 1.0)), 0.0)
    dinv2 = jnp.tile(dinv[:, None], (1, 128))

    x_pad = jnp.concatenate(
        [x.astype(F32), jnp.zeros((N_PAD - N, IN_DIM), F32)])
    batch_pad = jnp.concatenate(
        [batch.astype(I32), jnp.full((N_PAD - N,), G, I32)])
    batch3 = batch_pad.reshape(N_PAD // R, 1, R)
    g_list = [g0.reshape(1, H), g1.reshape(1, H), g2.reshape(1, H),
              g3.reshape(1, H)]
    bt_list = [bt0.reshape(1, H), bt1.reshape(1, H), bt2.reshape(1, H),
               bt3.reshape(1, H)]
    W_list = [W1, W2, W3]
    wv = Wout.reshape(1, H)
    bout2 = jnp.broadcast_to(bout.reshape(1, 1), (1, 128))
    zrows = jnp.zeros((N_PAD // N_TILES, SL), F32)

    prop_in = _make_prop(N_PAD, NS_IN, F_PAD)
    prop_h = _make_prop(N_PAD, NS_H, F_PAD)

    row_in = pl.BlockSpec((R, IN_DIM), lambda i: (i, 0))
    sl_in = pl.BlockSpec((NS_IN, R, SL), lambda i: (0, i, 0))
    sl_h = pl.BlockSpec((NS_H, R, SL), lambda i: (0, i, 0))
    dinv_spec = pl.BlockSpec((R, 128), lambda i: (i, 0))
    whole = lambda shape: pl.BlockSpec(shape, lambda i: tuple(0 for _ in shape))

    nb = N_PAD // R
    grid = (nb,)
    n_real = float(N)
    sds = jax.ShapeDtypeStruct

    xs = _row_call(
        functools.partial(_scale_body, NS_IN),
        sds((NS_IN, N_PAD, SL), F32), sl_in,
        [(x_pad, row_in), (dinv2, dinv_spec)], grid)
    q = prop_in(xs, srcg, dstg, zrows)
    q = _row_call(
        functools.partial(_mm_body, NS_IN, NS_H),
        sds((NS_H, N_PAD, SL), F32), sl_h,
        [(q, sl_in), (W0, whole((IN_DIM, H)))], grid)

    for l in range(3):
        stats = _row_call(
            functools.partial(_stats_body, NS_H),
            sds((2, H), F32), whole((2, H)),
            [(q, sl_h), (dinv2, dinv_spec)], grid)
        xw = _row_call(
            functools.partial(_bn_mm_body, n_real, NS_H, NS_H),
            sds((NS_H, N_PAD, SL), F32), sl_h,
            [(q, sl_h), (dinv2, dinv_spec), (stats, whole((2, H))),
             (g_list[l], whole((1, H))), (bt_list[l], whole((1, H))),
             (W_list[l], whole((H, H)))], grid)
        q = prop_h(xw, srcg, dstg, zrows)

    stats = _row_call(
        functools.partial(_stats_body, NS_H),
        sds((2, H), F32), whole((2, H)),
        [(q, sl_h), (dinv2, dinv_spec)], grid)
    res = pl.pallas_call(
        functools.partial(_final_body, n_real, nb, NS_H),
        grid=grid,
        in_specs=[sl_h, dinv_spec, whole((2, H)), whole((1, H)),
                  whole((1, H)), pl.BlockSpec((1, 1, R), lambda i: (i, 0, 0)),
                  whole((1, H)), whole((1, 128))],
        out_specs=whole((G, 128)),
        out_shape=sds((G, 128), F32),
        scratch_shapes=[pltpu.VMEM((G, H), F32)],
    )(q, dinv2, stats, g_list[3], bt_list[3], batch3, wv, bout2)
    return res[:, 0]


# speedup vs baseline: 3.5841x; 1.5085x over previous
import functools

import jax
import jax.numpy as jnp
from jax import lax
from jax.experimental import pallas as pl
from jax.experimental.pallas import tpu as pltpu
from jax.experimental.pallas import tpu_sc as plsc

F32 = jnp.float32
I32 = jnp.int32

N_CORES = 2
N_TILES = 16
EB = 128
SL = 128
R = 256
EPS = 1e-5
G = 64


def _split_store(o_ref, res, n_slices):
    for k in range(n_slices):
        o_ref[k] = res[:, k * SL:(k + 1) * SL]


def _cat_slices(q_ref, n_slices):
    return jnp.concatenate([q_ref[k] for k in range(n_slices)], axis=-1)


def _scale_body(n_slices, x_ref, dinv_ref, o_ref):
    _split_store(o_ref, x_ref[...] * dinv_ref[:, :1], n_slices)


def _mm_body(n_in, n_out, p_ref, w_ref, o_ref):
    h = _cat_slices(p_ref, n_in)
    res = jnp.dot(h, w_ref[...], preferred_element_type=F32)
    _split_store(o_ref, res, n_out)


def _stats_body(n_slices, q_ref, dinv_ref, o_ref):
    @pl.when(pl.program_id(0) == 0)
    def _():
        o_ref[...] = jnp.zeros_like(o_ref)

    c = _cat_slices(q_ref, n_slices) * dinv_ref[:, :1]
    s1 = jnp.sum(c, axis=0)
    s2 = jnp.sum(c * c, axis=0)
    o_ref[...] += jnp.concatenate([s1[None], s2[None]], axis=0)


def _gelu(h):
    return h * 0.5 * (1.0 + lax.erf(h * 0.7071067811865476))


def _bn_from_stats(c, stats_ref, g_ref, bt_ref, n_real):
    mean = stats_ref[0:1, :] / n_real
    var = stats_ref[1:2, :] / n_real - mean * mean
    xh = (c - mean) * lax.rsqrt(var + EPS)
    return xh * g_ref[...] + bt_ref[...]


def _bn_mm_body(n_real, n_in, n_out, q_ref, dinv_ref, stats_ref, g_ref,
                bt_ref, w_ref, o_ref):
    c = _cat_slices(q_ref, n_in) * dinv_ref[:, :1]
    h = _gelu(_bn_from_stats(c, stats_ref, g_ref, bt_ref, n_real))
    hp = h * dinv_ref[:, :1]
    res = jnp.dot(hp, w_ref[...], preferred_element_type=F32)
    _split_store(o_ref, res, n_out)


def _final_body(n_real, n_blocks, n_in, q_ref, dinv_ref, stats_ref, g_ref,
                bt_ref, batch_ref, wv_ref, bout_ref, o_ref, pool_acc):
    i = pl.program_id(0)

    @pl.when(i == 0)
    def _():
        pool_acc[...] = jnp.zeros_like(pool_acc)

    c = _cat_slices(q_ref, n_in) * dinv_ref[:, :1]
    h = _gelu(_bn_from_stats(c, stats_ref, g_ref, bt_ref, n_real))
    b = batch_ref[0]
    gids = lax.broadcasted_iota(I32, (G, R), 0)
    oh = (gids == b).astype(F32)
    pool_acc[...] += jnp.dot(oh, h, preferred_element_type=F32)

    @pl.when(i == n_blocks - 1)
    def _():
        fin = jnp.sum(pool_acc[...] * wv_ref[...], axis=1, keepdims=True)
        fin = fin + bout_ref[0:1, 0:1]
        o_ref[...] = jnp.broadcast_to(fin, (G, 128))


def _make_prop(n_pad, n_slices, f_pad):
    rpt = n_pad // N_TILES
    spc = n_slices // N_CORES
    ept = f_pad // N_TILES
    nb = ept // EB
    mesh = plsc.VectorSubcoreMesh(core_axis_name="c", subcore_axis_name="s",
                                  num_cores=N_CORES, num_subcores=N_TILES)

    @functools.partial(
        pl.kernel,
        out_type=jax.ShapeDtypeStruct((n_slices, n_pad, SL), F32),
        mesh=mesh,
        scratch_types=[
            pltpu.VMEM((2, EB), I32),
            pltpu.VMEM((2, EB), I32),
            pltpu.VMEM((EB, SL), F32),
            pltpu.VMEM((EB, SL), F32),
            pltpu.VMEM_SHARED((n_pad, SL), F32),
            pltpu.SemaphoreType.DMA,
            pltpu.SemaphoreType.DMA,
            pltpu.SemaphoreType.DMA,
            pltpu.SemaphoreType.DMA,
        ],
    )
    def prop(xw, edges, zrows, out,
             idx_a, idx_b, rows_a, rows_b, acc, sem_ga, sem_gb, sem_ia,
             sem_ib):
        core = lax.axis_index("c")
        tid = lax.axis_index("s")
        row0 = tid * nb

        def idx_start(buf, sem, b):
            return pltpu.async_copy(edges.at[row0 + b], buf, sem)

        def idx_wait(buf, sem, b):
            pltpu.make_async_copy(edges.at[row0 + b], buf, sem).wait()

        def gather_start(s, buf, rows, sem):
            return pltpu.async_copy(xw.at[s].at[buf.at[0]], rows, sem)

        def gather_wait(s, buf, rows, sem):
            pltpu.make_async_copy(xw.at[s].at[buf.at[0]], rows, sem).wait()

        for j in range(spc):
            s = core * spc + j
            pltpu.sync_copy(zrows, acc.at[pl.ds(tid * rpt, rpt)])
            plsc.subcore_barrier()

            idx_start(idx_a, sem_ia, 0)
            idx_wait(idx_a, sem_ia, 0)
            gather_start(s, idx_a, rows_a, sem_ga)
            idx_start(idx_b, sem_ib, 1)

            def body(b2, _, s=s):
                b = b2 * 2
                idx_wait(idx_b, sem_ib, b + 1)
                gather_start(s, idx_b, rows_b, sem_gb)
                gather_wait(s, idx_a, rows_a, sem_ga)
                pltpu.sync_copy(rows_a, acc.at[idx_a.at[1]], add=True)

                @pl.when(b + 2 < nb)
                def _():
                    idx_start(idx_a, sem_ia, b + 2)
                    idx_wait(idx_a, sem_ia, b + 2)
                    gather_start(s, idx_a, rows_a, sem_ga)

                gather_wait(s, idx_b, rows_b, sem_gb)
                pltpu.sync_copy(rows_b, acc.at[idx_b.at[1]], add=True)

                @pl.when(b + 2 < nb)
                def _():
                    idx_start(idx_b, sem_ib, b + 3)

                return 0

            lax.fori_loop(0, nb // 2, body, 0)
            plsc.subcore_barrier()
            pltpu.sync_copy(acc.at[pl.ds(tid * rpt, rpt)],
                            out.at[s].at[pl.ds(tid * rpt, rpt)])
            plsc.subcore_barrier()

    return prop


def _row_call(body, out_sds, out_spec, args_specs, grid, scratch=None):
    in_specs = [s for _, s in args_specs]
    args = [a for a, _ in args_specs]
    return pl.pallas_call(
        body, grid=grid, in_specs=in_specs, out_specs=out_spec,
        out_shape=out_sds, scratch_shapes=scratch or [],
    )(*args)


def kernel(x, edge_index, batch, W0, b0, g0, bt0, W1, b1, g1, bt1,
           W2, b2, g2, bt2, W3, b3, g3, bt3, Wout, bout):
    N, IN_DIM = x.shape
    H = W0.shape[1]
    E = edge_index.shape[1]
    F = E + N
    N_PAD = ((N + 1023) // 1024) * 1024
    F_PAD = ((F + 2 * N_TILES * EB - 1) // (2 * N_TILES * EB)) * (
        2 * N_TILES * EB)
    ZROW = N
    NS_IN = IN_DIM // SL
    NS_H = H // SL

    src = edge_index[0].astype(I32)
    dst = edge_index[1].astype(I32)
    loop = jnp.arange(N, dtype=I32)
    srcg = jnp.concatenate(
        [src, loop, jnp.full((F_PAD - F,), ZROW, I32)])
    dstg = jnp.concatenate(
        [dst, loop, jnp.full((F_PAD - F,), ZROW, I32)])
    edges = jnp.stack([srcg.reshape(-1, EB), dstg.reshape(-1, EB)], axis=1)
    deg = jnp.zeros((N_PAD,), F32).at[dstg].add(1.0)
    deg = deg.at[ZROW].set(0.0)
    dinv = jnp.where(deg > 0, lax.rsqrt(jnp.maximum(deg, 1.0)), 0.0)
    dinv2 = jnp.tile(dinv[:, None], (1, 128))

    x_pad = jnp.concatenate(
        [x.astype(F32), jnp.zeros((N_PAD - N, IN_DIM), F32)])
    batch_pad = jnp.concatenate(
        [batch.astype(I32), jnp.full((N_PAD - N,), G, I32)])
    batch3 = batch_pad.reshape(N_PAD // R, 1, R)
    g_list = [g0.reshape(1, H), g1.reshape(1, H), g2.reshape(1, H),
              g3.reshape(1, H)]
    bt_list = [bt0.reshape(1, H), bt1.reshape(1, H), bt2.reshape(1, H),
               bt3.reshape(1, H)]
    W_list = [W1, W2, W3]
    wv = Wout.reshape(1, H)
    bout2 = jnp.broadcast_to(bout.reshape(1, 1), (1, 128))
    zrows = jnp.zeros((N_PAD // N_TILES, SL), F32)

    prop_in = _make_prop(N_PAD, NS_IN, F_PAD)
    prop_h = _make_prop(N_PAD, NS_H, F_PAD)

    row_in = pl.BlockSpec((R, IN_DIM), lambda i: (i, 0))
    sl_in = pl.BlockSpec((NS_IN, R, SL), lambda i: (0, i, 0))
    sl_h = pl.BlockSpec((NS_H, R, SL), lambda i: (0, i, 0))
    dinv_spec = pl.BlockSpec((R, 128), lambda i: (i, 0))
    whole = lambda shape: pl.BlockSpec(shape, lambda i: tuple(0 for _ in shape))

    nb = N_PAD // R
    grid = (nb,)
    n_real = float(N)
    sds = jax.ShapeDtypeStruct

    xs = _row_call(
        functools.partial(_scale_body, NS_IN),
        sds((NS_IN, N_PAD, SL), F32), sl_in,
        [(x_pad, row_in), (dinv2, dinv_spec)], grid)
    q = prop_in(xs, edges, zrows)
    q = _row_call(
        functools.partial(_mm_body, NS_IN, NS_H),
        sds((NS_H, N_PAD, SL), F32), sl_h,
        [(q, sl_in), (W0, whole((IN_DIM, H)))], grid)

    for l in range(3):
        stats = _row_call(
            functools.partial(_stats_body, NS_H),
            sds((2, H), F32), whole((2, H)),
            [(q, sl_h), (dinv2, dinv_spec)], grid)
        xw = _row_call(
            functools.partial(_bn_mm_body, n_real, NS_H, NS_H),
            sds((NS_H, N_PAD, SL), F32), sl_h,
            [(q, sl_h), (dinv2, dinv_spec), (stats, whole((2, H))),
             (g_list[l], whole((1, H))), (bt_list[l], whole((1, H))),
             (W_list[l], whole((H, H)))], grid)
        q = prop_h(xw, edges, zrows)

    stats = _row_call(
        functools.partial(_stats_body, NS_H),
        sds((2, H), F32), whole((2, H)),
        [(q, sl_h), (dinv2, dinv_spec)], grid)
    res = pl.pallas_call(
        functools.partial(_final_body, n_real, nb, NS_H),
        grid=grid,
        in_specs=[sl_h, dinv_spec, whole((2, H)), whole((1, H)),
                  whole((1, H)), pl.BlockSpec((1, 1, R), lambda i: (i, 0, 0)),
                  whole((1, H)), whole((1, 128))],
        out_specs=whole((G, 128)),
        out_shape=sds((G, 128), F32),
        scratch_shapes=[pltpu.VMEM((G, H), F32)],
    )(q, dinv2, stats, g_list[3], bt_list[3], batch3, wv, bout2)
    return res[:, 0]
